# baseline (speedup 1.0000x reference)
import functools

import jax
import jax.numpy as jnp
from jax import lax
from jax.experimental import pallas as pl
from jax.experimental.pallas import tpu as pltpu
from jax.experimental.pallas import tpu_sc as plsc

NC = 2
NS = 16
L = 16
NW = NC * NS

N = 16384 * 26
D = 16
R = N // NW
C = 1024
NCH = R // C
G = 128
NG = C // G

_SPLAT_DNUMS = lax.GatherDimensionNumbers(
    offset_dims=(), collapsed_slice_dims=(0,), start_index_map=(0,)
)


def _splat(vec, i):
    return lax.gather(
        vec,
        jnp.full((L, 1), i, jnp.int32),
        _SPLAT_DNUMS,
        (1,),
        mode=lax.GatherScatterMode.PROMISE_IN_BOUNDS,
    )


def _sc_embed(table, idx_flat, val_flat):
    mesh = plsc.VectorSubcoreMesh(core_axis_name="c", subcore_axis_name="s")

    @functools.partial(
        pl.kernel,
        out_type=jax.ShapeDtypeStruct((N, D), jnp.float32),
        mesh=mesh,
        compiler_params=pltpu.CompilerParams(use_tc_tiling_on_sc=False),
        scratch_types=[
            pltpu.VMEM((R,), jnp.int32),
            pltpu.VMEM((R,), jnp.float32),
            pltpu.VMEM((C, D), jnp.float32),
            pltpu.SemaphoreType.DMA,
        ],
    )
    def body(tab_hbm, idx_hbm, val_hbm, out_hbm, idx_v, val_v, rows_v, sem):
        wid = lax.axis_index("s") * NC + lax.axis_index("c")
        base = wid * R
        pltpu.sync_copy(idx_hbm.at[pl.ds(base, R)], idx_v)
        pltpu.sync_copy(val_hbm.at[pl.ds(base, R)], val_v)

        def chunk_body(c, carry):
            off = c * C
            cps = [
                pltpu.async_copy(
                    tab_hbm.at[idx_v.at[pl.ds(off + k * G, G)]],
                    rows_v.at[pl.ds(k * G, G)],
                    sem,
                )
                for k in range(NG)
            ]
            for cp in cps:
                cp.wait()

            def grp_body(g, carry2):
                vals16 = val_v[pl.ds(off + g * L, L)]
                r0 = g * L
                for i in range(L):
                    rows_v[r0 + i, :] = rows_v[r0 + i, :] * _splat(vals16, i)
                return carry2

            lax.fori_loop(0, C // L, grp_body, 0)
            pltpu.sync_copy(rows_v, out_hbm.at[pl.ds(base + off, C)])
            return carry

        lax.fori_loop(0, NCH, chunk_body, 0)

    return body(table, idx_flat, val_flat)


def kernel(ids, value, table):
    idx_flat = ids.reshape(-1).astype(jnp.int32)
    val_flat = value.reshape(-1)
    out = _sc_embed(table, idx_flat, val_flat)
    return out.reshape(ids.shape[0], ids.shape[1], D)


# speedup vs baseline: 1.5929x; 1.5929x over previous
import functools

import jax
import jax.numpy as jnp
from jax import lax
from jax.experimental import pallas as pl
from jax.experimental.pallas import tpu as pltpu
from jax.experimental.pallas import tpu_sc as plsc

NC = 2
NS = 16
L = 16
NW = NC * NS

B = 16384
F = 26
D = 16
BW = B // NW
NBLK = BW // 128


def _sc_embed(table, ids3, valT):
    mesh = plsc.VectorSubcoreMesh(core_axis_name="c", subcore_axis_name="s")

    @functools.partial(
        pl.kernel,
        out_type=jax.ShapeDtypeStruct((F, D, B), jnp.float32),
        mesh=mesh,
        compiler_params=pltpu.CompilerParams(
            use_tc_tiling_on_sc=False, needs_layout_passes=False
        ),
        scratch_types=[
            pltpu.VMEM((F, NBLK, 128), jnp.int32),
            pltpu.VMEM((F, BW), jnp.float32),
            pltpu.VMEM((2, BW, D), jnp.float32),
            pltpu.VMEM((2, D, BW), jnp.float32),
            pltpu.SemaphoreType.DMA,
            pltpu.SemaphoreType.DMA,
        ],
    )
    def body(tab_hbm, ids_hbm, val_hbm, out_hbm,
             idx_v, val_v, rows_v, stage_v, sem_g, sem_o):
        wid = lax.axis_index("s") * NC + lax.axis_index("c")
        b0 = wid * BW
        pltpu.sync_copy(ids_hbm.at[:, pl.ds(wid * NBLK, NBLK), :], idx_v)
        pltpu.sync_copy(val_hbm.at[:, pl.ds(b0, BW)], val_v)

        lanes = lax.iota(jnp.int32, L)

        def fire_gathers(f, p):
            return [
                pltpu.async_copy(
                    tab_hbm.at[idx_v.at[f, blk]],
                    rows_v.at[p, pl.ds(blk * 128, 128)],
                    sem_g,
                )
                for blk in range(NBLK)
            ]

        def compute_field(f, p):
            pconst = jnp.full((L,), p, jnp.int32)

            def grp(g, carry):
                valv = val_v[f, pl.ds(g * L, L)]
                ridx = lanes + g * L
                for e in range(D):
                    vec = plsc.load_gather(
                        rows_v, [pconst, ridx, jnp.full((L,), e, jnp.int32)]
                    )
                    stage_v[p, e, pl.ds(g * L, L)] = vec * valv
                return carry

            lax.fori_loop(0, BW // L, grp, 0)

        gacc = {0: fire_gathers(0, 0)}
        oacc = {}
        for f in range(F):
            p = f % 2
            if f + 1 < F:
                gacc[f + 1] = fire_gathers(f + 1, (f + 1) % 2)
            for cp in gacc[f]:
                cp.wait()
            compute_field(f, p)
            if f >= 2:
                oacc[f - 2].wait()
            oacc[f] = pltpu.async_copy(
                stage_v.at[p], out_hbm.at[f, :, pl.ds(b0, BW)], sem_o
            )
        oacc[F - 2].wait()
        oacc[F - 1].wait()

    return body(table, ids3, valT)


def kernel(ids, value, table):
    ids3 = ids.T.astype(jnp.int32).reshape(F, B // 128, 128)
    valT = value.T
    out = _sc_embed(table, ids3, valT)
    return out.transpose(2, 0, 1)


# speedup vs baseline: 1.6789x; 1.0540x over previous
import functools

import jax
import jax.numpy as jnp
from jax import lax
from jax.experimental import pallas as pl
from jax.experimental.pallas import tpu as pltpu
from jax.experimental.pallas import tpu_sc as plsc

NC = 2
NS = 16
L = 16
NW = NC * NS

B = 16384
F = 26
D = 16
BW = B // NW
NBLK = BW // 128
TILE = 8 * 128
WTW = NBLK * TILE


def _sc_embed(table, ids3, valT):
    mesh = plsc.VectorSubcoreMesh(core_axis_name="c", subcore_axis_name="s")

    @functools.partial(
        pl.kernel,
        out_type=jax.ShapeDtypeStruct((F, 2, (B // 128) * TILE), jnp.float32),
        mesh=mesh,
        compiler_params=pltpu.CompilerParams(
            use_tc_tiling_on_sc=False, needs_layout_passes=False
        ),
        scratch_types=[
            pltpu.VMEM((F, NBLK, 128), jnp.int32),
            pltpu.VMEM((F, BW), jnp.float32),
            pltpu.VMEM((2, BW, D), jnp.float32),
            pltpu.VMEM((2, 2, WTW), jnp.float32),
            pltpu.SemaphoreType.DMA,
            pltpu.SemaphoreType.DMA,
        ],
    )
    def body(tab_hbm, ids_hbm, val_hbm, out_hbm,
             idx_v, val_v, rows_v, stage_v, sem_g, sem_o):
        wid = lax.axis_index("s") * NC + lax.axis_index("c")
        b0 = wid * BW
        pltpu.sync_copy(ids_hbm.at[:, pl.ds(wid * NBLK, NBLK), :], idx_v)
        pltpu.sync_copy(val_hbm.at[:, pl.ds(b0, BW)], val_v)

        lanes = lax.iota(jnp.int32, L)

        def fire_gathers(f, p):
            return [
                pltpu.async_copy(
                    tab_hbm.at[idx_v.at[f, blk]],
                    rows_v.at[p, pl.ds(blk * 128, 128)],
                    sem_g,
                )
                for blk in range(NBLK)
            ]

        def compute_field(f, p):
            pconst = jnp.full((L,), p, jnp.int32)

            def grp(g, carry):
                valv = val_v[f, pl.ds(g * L, L)]
                ridx = lanes + g * L
                off = (g // 8) * TILE + (g % 8) * L
                for e in range(D):
                    vec = plsc.load_gather(
                        rows_v, [pconst, ridx, jnp.full((L,), e, jnp.int32)]
                    )
                    stage_v[p, e // 8, pl.ds(off + (e % 8) * 128, L)] = vec * valv
                return carry

            lax.fori_loop(0, BW // L, grp, 0)

        gacc = {0: fire_gathers(0, 0)}
        oacc = {}
        for f in range(F):
            p = f % 2
            if f + 1 < F:
                gacc[f + 1] = fire_gathers(f + 1, (f + 1) % 2)
            for cp in gacc[f]:
                cp.wait()
            compute_field(f, p)
            if f >= 2:
                for cp in oacc[f - 2]:
                    cp.wait()
            oacc[f] = [
                pltpu.async_copy(
                    stage_v.at[p, s],
                    out_hbm.at[f, s, pl.ds(wid * WTW, WTW)],
                    sem_o,
                )
                for s in range(2)
            ]
        for f in (F - 2, F - 1):
            for cp in oacc[f]:
                cp.wait()

    return body(table, ids3, valT)


def kernel(ids, value, table):
    ids3 = ids.T.astype(jnp.int32).reshape(F, B // 128, 128)
    valT = value.T
    out = _sc_embed(table, ids3, valT)
    out5 = out.reshape(F, 2, B // 128, 8, 128)
    return out5.transpose(2, 4, 0, 1, 3).reshape(B, F, D)
